# baseline (device time: 422152 ns/iter reference)
def kernel(x, dest):
    import jax
    import jax.numpy as jnp
    import numpy as np
    from jax import lax
    from jax.experimental import pallas as pl
    from jax.experimental.pallas import tpu as pltpu

    N = 32
    H_R = 16
    H_L = 15
    M, K = x.shape
    NSUB = 4
    HM = M // NSUB
    d2 = dest.reshape(8, 128)

    perm = np.arange(N, dtype=np.int32)
    try:
        import distributed_mesh_v7x as dm

        mesh_coords = [
            tuple(d.coords) for d in dm.get_mesh("i", world_size=N).devices.flat
        ]
        xs = sorted({c[0] for c in mesh_coords})
        ys = sorted({c[1] for c in mesh_coords})
        zs = sorted({c[2] for c in mesh_coords})
        if len(xs) == 2 and len(ys) * len(zs) * 2 == N:
            half = []
            for yi, y in enumerate(ys):
                zo = zs if yi % 2 == 0 else zs[::-1]
                half += [(xs[0], y, z) for z in zo]
            cycle = half + [(xs[1], y, z) for (_, y, z) in reversed(half)]
            single_hop = all(
                sum(abs(a - b) for a, b in zip(cycle[t], cycle[(t + 1) % N])) == 1
                for t in range(N)
            )
            if single_hop and set(cycle) == set(mesh_coords):
                midx = {c: i for i, c in enumerate(mesh_coords)}
                perm = np.array([midx[c] for c in cycle], dtype=np.int32)
    except Exception:
        pass
    inv_perm = np.argsort(perm).astype(np.int32)

    perm_a = jnp.asarray(perm)
    my = lax.axis_index("i")
    p = jnp.asarray(inv_perm)[my]
    right = perm_a[(p + 1) % N]
    left = perm_a[(p + N - 1) % N]
    o_r = perm_a[(p - jnp.arange(H_R, dtype=jnp.int32) + N) % N]
    o_l = perm_a[(p + jnp.arange(H_L, dtype=jnp.int32)) % N]
    meta = jnp.concatenate([right[None], left[None], o_r, o_l]).astype(jnp.int32)

    def body(
        x_ref,
        d_ref,
        meta_ref,
        xg_ref,
        dg_ref,
        cp_sems,
        sx_r,
        rx_r,
        sd_r,
        rd_r,
        sx_l,
        rx_l,
        sd_l,
        rd_l,
    ):
        my_in = lax.axis_index("i")
        rgt = meta_ref[0]
        lft = meta_ref[1]

        own_x = pltpu.make_async_copy(x_ref, xg_ref.at[my_in], cp_sems.at[0])
        own_d = pltpu.make_async_copy(d_ref, dg_ref.at[my_in], cp_sems.at[1])
        own_x.start()
        own_d.start()

        def rd_half(side, h, k):
            if side == "r":
                org, dev, ss, rs = meta_ref[2 + h - 1], rgt, sx_r, rx_r
            else:
                org, dev, ss, rs = meta_ref[2 + H_R + h - 1], lft, sx_l, rx_l
            src = x_ref if h == 1 else xg_ref.at[org]
            return pltpu.make_async_remote_copy(
                src.at[pl.ds(k * HM, HM)],
                xg_ref.at[org, pl.ds(k * HM, HM)],
                ss.at[h - 1, k],
                rs.at[h - 1, k],
                device_id=(dev,),
                device_id_type=pl.DeviceIdType.MESH,
            )

        def rd_dest(side, h):
            if side == "r":
                org, dev, ss, rs = meta_ref[2 + h - 1], rgt, sd_r, rd_r
            else:
                org, dev, ss, rs = meta_ref[2 + H_R + h - 1], lft, sd_l, rd_l
            src = d_ref if h == 1 else dg_ref.at[org]
            return pltpu.make_async_remote_copy(
                src,
                dg_ref.at[org],
                ss.at[h - 1],
                rs.at[h - 1],
                device_id=(dev,),
                device_id_type=pl.DeviceIdType.MESH,
            )

        for k in range(NSUB):
            rd_half("r", 1, k).start()
            rd_half("l", 1, k).start()
        rd_dest("r", 1).start()
        rd_dest("l", 1).start()

        for h in range(2, H_R + 1):
            for k in range(NSUB):
                rd_half("r", h - 1, k).wait_recv()
                rd_half("r", h, k).start()
                if h <= H_L:
                    rd_half("l", h - 1, k).wait_recv()
                    rd_half("l", h, k).start()
            rd_dest("r", h - 1).wait_recv()
            rd_dest("r", h).start()
            if h <= H_L:
                rd_dest("l", h - 1).wait_recv()
                rd_dest("l", h).start()

        for k in range(NSUB):
            rd_half("r", H_R, k).wait_recv()
            rd_half("l", H_L, k).wait_recv()
        rd_dest("r", H_R).wait_recv()
        rd_dest("l", H_L).wait_recv()
        for h in range(1, H_R + 1):
            for k in range(NSUB):
                rd_half("r", h, k).wait_send()
        for h in range(1, H_L + 1):
            for k in range(NSUB):
                rd_half("l", h, k).wait_send()
        for h in range(1, H_R + 1):
            rd_dest("r", h).wait_send()
        for h in range(1, H_L + 1):
            rd_dest("l", h).wait_send()

        own_x.wait()
        own_d.wait()

    xg, dg = pl.pallas_call(
        body,
        out_shape=[
            jax.ShapeDtypeStruct((N, M, K), jnp.float32),
            jax.ShapeDtypeStruct((N, 8, 128), jnp.int32),
        ],
        in_specs=[
            pl.BlockSpec(memory_space=pl.ANY),
            pl.BlockSpec(memory_space=pl.ANY),
            pl.BlockSpec(memory_space=pltpu.MemorySpace.SMEM),
        ],
        out_specs=[
            pl.BlockSpec(memory_space=pl.ANY),
            pl.BlockSpec(memory_space=pl.ANY),
        ],
        scratch_shapes=[
            pltpu.SemaphoreType.DMA((2,)),
            pltpu.SemaphoreType.DMA((H_R, NSUB)),
            pltpu.SemaphoreType.DMA((H_R, NSUB)),
            pltpu.SemaphoreType.DMA((H_R,)),
            pltpu.SemaphoreType.DMA((H_R,)),
            pltpu.SemaphoreType.DMA((H_L, NSUB)),
            pltpu.SemaphoreType.DMA((H_L, NSUB)),
            pltpu.SemaphoreType.DMA((H_L,)),
            pltpu.SemaphoreType.DMA((H_L,)),
        ],
    )(x, d2, meta)

    gdest = dg.reshape(N * M)
    pos = jnp.cumsum((gdest == my).astype(jnp.int32))
    idx = jnp.searchsorted(
        pos, jnp.arange(1, M + 1, dtype=jnp.int32), method="compare_all"
    )
    return xg.reshape(N * M, K)[idx]


# device time: 414781 ns/iter; 1.0178x vs baseline; 1.0178x over previous
def kernel(x, dest):
    import jax
    import jax.numpy as jnp
    import numpy as np
    from jax import lax
    from jax.experimental import pallas as pl
    from jax.experimental.pallas import tpu as pltpu

    N = 32
    H_R = 16
    H_L = 15
    M, K = x.shape
    NSUB = 4
    HM = M // NSUB
    d2 = dest.reshape(8, 128)

    perm = np.arange(N, dtype=np.int32)
    try:
        import distributed_mesh_v7x as dm

        mesh_coords = [
            tuple(d.coords) for d in dm.get_mesh("i", world_size=N).devices.flat
        ]
        xs = sorted({c[0] for c in mesh_coords})
        ys = sorted({c[1] for c in mesh_coords})
        zs = sorted({c[2] for c in mesh_coords})
        if len(xs) == 2 and len(ys) * len(zs) * 2 == N:
            half = []
            for yi, y in enumerate(ys):
                zo = zs if yi % 2 == 0 else zs[::-1]
                half += [(xs[0], y, z) for z in zo]
            cycle = half + [(xs[1], y, z) for (_, y, z) in reversed(half)]
            single_hop = all(
                sum(abs(a - b) for a, b in zip(cycle[t], cycle[(t + 1) % N])) == 1
                for t in range(N)
            )
            if single_hop and set(cycle) == set(mesh_coords):
                midx = {c: i for i, c in enumerate(mesh_coords)}
                perm = np.array([midx[c] for c in cycle], dtype=np.int32)
    except Exception:
        pass
    inv_perm = np.argsort(perm).astype(np.int32)

    perm_a = jnp.asarray(perm)
    my = lax.axis_index("i")
    p = jnp.asarray(inv_perm)[my]
    right = perm_a[(p + 1) % N]
    left = perm_a[(p + N - 1) % N]
    o_r = perm_a[(p - jnp.arange(H_R, dtype=jnp.int32) + N) % N]
    o_l = perm_a[(p + jnp.arange(H_L, dtype=jnp.int32)) % N]
    meta = jnp.concatenate([right[None], left[None], o_r, o_l]).astype(jnp.int32)

    def body(
        x_ref,
        d_ref,
        meta_ref,
        xg_ref,
        dg_ref,
        cp_sems,
        sx_r,
        rx_r,
        sd_r,
        rd_r,
        sx_l,
        rx_l,
        sd_l,
        rd_l,
    ):
        my_in = lax.axis_index("i")
        rgt = meta_ref[0]
        lft = meta_ref[1]

        barrier_sem = pltpu.get_barrier_semaphore()
        for nbr in (lft, rgt):
            pl.semaphore_signal(
                barrier_sem,
                inc=1,
                device_id=(nbr,),
                device_id_type=pl.DeviceIdType.MESH,
            )
        pl.semaphore_wait(barrier_sem, 2)

        own_x = pltpu.make_async_copy(x_ref, xg_ref.at[my_in], cp_sems.at[0])
        own_d = pltpu.make_async_copy(d_ref, dg_ref.at[my_in], cp_sems.at[1])
        own_x.start()
        own_d.start()

        def rd_half(side, h, k):
            if side == "r":
                org, dev, ss, rs = meta_ref[2 + h - 1], rgt, sx_r, rx_r
            else:
                org, dev, ss, rs = meta_ref[2 + H_R + h - 1], lft, sx_l, rx_l
            src = x_ref if h == 1 else xg_ref.at[org]
            return pltpu.make_async_remote_copy(
                src.at[pl.ds(k * HM, HM)],
                xg_ref.at[org, pl.ds(k * HM, HM)],
                ss.at[h - 1, k],
                rs.at[h - 1, k],
                device_id=(dev,),
                device_id_type=pl.DeviceIdType.MESH,
            )

        def rd_dest(side, h):
            if side == "r":
                org, dev, ss, rs = meta_ref[2 + h - 1], rgt, sd_r, rd_r
            else:
                org, dev, ss, rs = meta_ref[2 + H_R + h - 1], lft, sd_l, rd_l
            src = d_ref if h == 1 else dg_ref.at[org]
            return pltpu.make_async_remote_copy(
                src,
                dg_ref.at[org],
                ss.at[h - 1],
                rs.at[h - 1],
                device_id=(dev,),
                device_id_type=pl.DeviceIdType.MESH,
            )

        for k in range(NSUB):
            rd_half("r", 1, k).start()
            rd_half("l", 1, k).start()
        rd_dest("r", 1).start()
        rd_dest("l", 1).start()

        for h in range(2, H_R + 1):
            for k in range(NSUB):
                rd_half("r", h - 1, k).wait_recv()
                rd_half("r", h, k).start()
                if h <= H_L:
                    rd_half("l", h - 1, k).wait_recv()
                    rd_half("l", h, k).start()
            rd_dest("r", h - 1).wait_recv()
            rd_dest("r", h).start()
            if h <= H_L:
                rd_dest("l", h - 1).wait_recv()
                rd_dest("l", h).start()

        for k in range(NSUB):
            rd_half("r", H_R, k).wait_recv()
            rd_half("l", H_L, k).wait_recv()
        rd_dest("r", H_R).wait_recv()
        rd_dest("l", H_L).wait_recv()
        for h in range(1, H_R + 1):
            for k in range(NSUB):
                rd_half("r", h, k).wait_send()
        for h in range(1, H_L + 1):
            for k in range(NSUB):
                rd_half("l", h, k).wait_send()
        for h in range(1, H_R + 1):
            rd_dest("r", h).wait_send()
        for h in range(1, H_L + 1):
            rd_dest("l", h).wait_send()

        own_x.wait()
        own_d.wait()

    xg, dg = pl.pallas_call(
        body,
        out_shape=[
            jax.ShapeDtypeStruct((N, M, K), jnp.float32),
            jax.ShapeDtypeStruct((N, 8, 128), jnp.int32),
        ],
        in_specs=[
            pl.BlockSpec(memory_space=pl.ANY),
            pl.BlockSpec(memory_space=pl.ANY),
            pl.BlockSpec(memory_space=pltpu.MemorySpace.SMEM),
        ],
        out_specs=[
            pl.BlockSpec(memory_space=pl.ANY),
            pl.BlockSpec(memory_space=pl.ANY),
        ],
        scratch_shapes=[
            pltpu.SemaphoreType.DMA((2,)),
            pltpu.SemaphoreType.DMA((H_R, NSUB)),
            pltpu.SemaphoreType.DMA((H_R, NSUB)),
            pltpu.SemaphoreType.DMA((H_R,)),
            pltpu.SemaphoreType.DMA((H_R,)),
            pltpu.SemaphoreType.DMA((H_L, NSUB)),
            pltpu.SemaphoreType.DMA((H_L, NSUB)),
            pltpu.SemaphoreType.DMA((H_L,)),
            pltpu.SemaphoreType.DMA((H_L,)),
        ],
        compiler_params=pltpu.CompilerParams(collective_id=0),
    )(x, d2, meta)

    gdest = dg.reshape(N * M)
    pos = jnp.cumsum((gdest == my).astype(jnp.int32))
    idx = jnp.searchsorted(
        pos, jnp.arange(1, M + 1, dtype=jnp.int32), method="compare_all"
    )
    return xg.reshape(N * M, K)[idx]
